# baseline (speedup 1.0000x reference)
import functools

import jax
import jax.numpy as jnp
from jax import lax
from jax.experimental import pallas as pl
from jax.experimental.pallas import tpu as pltpu
from jax.experimental.pallas import tpu_sc as plsc

S = 200
B = 4096
D = 64
O = 2
NW = 32
B_PER_W = B // NW
HALF_S = S // 2


def _sc_body(xT_hbm, table_hbm, pooled_hbm, xchunk_v, buf, pooled_v, sem):
    wid = lax.axis_index("s") * 2 + lax.axis_index("c")
    base = wid * B_PER_W

    pltpu.sync_copy(xT_hbm.at[pl.ds(base * 2, 2 * B_PER_W)], xchunk_v)

    def per_col(i, carry):
        c0 = pltpu.async_copy(table_hbm.at[xchunk_v.at[2 * i]],
                              buf.at[pl.ds(0, HALF_S)], sem)
        c1 = pltpu.async_copy(table_hbm.at[xchunk_v.at[2 * i + 1]],
                              buf.at[pl.ds(HALF_S, HALF_S)], sem)
        c0.wait()
        c1.wait()

        def red(s, accs):
            a0, a1, a2, a3 = accs
            a0 = a0 + buf[s, pl.ds(0, 16)]
            a1 = a1 + buf[s, pl.ds(16, 16)]
            a2 = a2 + buf[s, pl.ds(32, 16)]
            a3 = a3 + buf[s, pl.ds(48, 16)]
            return a0, a1, a2, a3

        z = jnp.zeros((16,), jnp.float32)
        a0, a1, a2, a3 = lax.fori_loop(0, S, red, (z, z, z, z))
        pooled_v[i, pl.ds(0, 16)] = a0
        pooled_v[i, pl.ds(16, 16)] = a1
        pooled_v[i, pl.ds(32, 16)] = a2
        pooled_v[i, pl.ds(48, 16)] = a3
        return carry

    lax.fori_loop(0, B_PER_W, per_col, 0)

    pltpu.sync_copy(pooled_v, pooled_hbm.at[pl.ds(base, B_PER_W)])


@jax.jit
def _fast_text(xT, emb_table, w_scaled, b_row):
    mesh = plsc.VectorSubcoreMesh(core_axis_name="c", subcore_axis_name="s")
    pooled = functools.partial(
        pl.kernel,
        out_type=jax.ShapeDtypeStruct((B, D), jnp.float32),
        mesh=mesh,
        compiler_params=pltpu.CompilerParams(use_tc_tiling_on_sc=False),
        scratch_types=[
            pltpu.VMEM((2 * B_PER_W, HALF_S), jnp.int32),
            pltpu.VMEM((S, D), jnp.float32),
            pltpu.VMEM((B_PER_W, D), jnp.float32),
            pltpu.SemaphoreType.DMA,
        ],
    )(_sc_body)(xT, emb_table)

    def _fc_body(p_ref, w_ref, b_ref, o_ref):
        o_ref[...] = (
            jnp.dot(p_ref[...], w_ref[...],
                    preferred_element_type=jnp.float32)
            + b_ref[...]
        )

    return pl.pallas_call(
        _fc_body,
        out_shape=jax.ShapeDtypeStruct((B, O), jnp.float32),
    )(pooled, w_scaled, b_row)


def kernel(x, emb_table, fc_w, fc_b):
    xT = x.astype(jnp.int32).T.reshape(2 * B, HALF_S)
    w_scaled = (fc_w.astype(jnp.float32) / S).T
    b_row = fc_b.astype(jnp.float32)[None, :]
    return _fast_text(xT, emb_table, w_scaled, b_row)


# speedup vs baseline: 1.1707x; 1.1707x over previous
import functools

import jax
import jax.numpy as jnp
from jax import lax
from jax.experimental import pallas as pl
from jax.experimental.pallas import tpu as pltpu
from jax.experimental.pallas import tpu_sc as plsc

S = 200
B = 4096
D = 64
O = 2
NW = 32
B_PER_W = B // NW
HALF_S = S // 2


def _sc_body(xT_hbm, table_hbm, pooled_hbm, xchunk_v, buf_a, buf_b,
             pooled_v, sem_a, sem_b):
    wid = lax.axis_index("s") * 2 + lax.axis_index("c")
    base = wid * B_PER_W

    pltpu.sync_copy(xT_hbm.at[pl.ds(base * 2, 2 * B_PER_W)], xchunk_v)

    def issue(col, buf, sem):
        c0 = pltpu.async_copy(table_hbm.at[xchunk_v.at[2 * col]],
                              buf.at[pl.ds(0, HALF_S)], sem)
        c1 = pltpu.async_copy(table_hbm.at[xchunk_v.at[2 * col + 1]],
                              buf.at[pl.ds(HALF_S, HALF_S)], sem)
        return c0, c1

    def drain(buf, sem):
        pltpu.make_async_copy(table_hbm.at[xchunk_v.at[0]],
                              buf.at[pl.ds(0, HALF_S)], sem).wait()
        pltpu.make_async_copy(table_hbm.at[xchunk_v.at[1]],
                              buf.at[pl.ds(HALF_S, HALF_S)], sem).wait()

    def reduce_into(col, buf):
        def red(s, accs):
            a0, a1, a2, a3 = accs
            a0 = a0 + buf[s, pl.ds(0, 16)]
            a1 = a1 + buf[s, pl.ds(16, 16)]
            a2 = a2 + buf[s, pl.ds(32, 16)]
            a3 = a3 + buf[s, pl.ds(48, 16)]
            return a0, a1, a2, a3

        z = jnp.zeros((16,), jnp.float32)
        a0, a1, a2, a3 = lax.fori_loop(0, S, red, (z, z, z, z), unroll=8)
        pooled_v[col, pl.ds(0, 16)] = a0
        pooled_v[col, pl.ds(16, 16)] = a1
        pooled_v[col, pl.ds(32, 16)] = a2
        pooled_v[col, pl.ds(48, 16)] = a3

    issue(0, buf_a, sem_a)

    def per_pair(i, carry):
        issue(2 * i + 1, buf_b, sem_b)
        drain(buf_a, sem_a)
        reduce_into(2 * i, buf_a)

        @pl.when(i < B_PER_W // 2 - 1)
        def _():
            issue(2 * i + 2, buf_a, sem_a)

        drain(buf_b, sem_b)
        reduce_into(2 * i + 1, buf_b)
        return carry

    lax.fori_loop(0, B_PER_W // 2, per_pair, 0)

    pltpu.sync_copy(pooled_v, pooled_hbm.at[pl.ds(base, B_PER_W)])


@jax.jit
def _fast_text(xT, emb_table, w_scaled, b_row):
    mesh = plsc.VectorSubcoreMesh(core_axis_name="c", subcore_axis_name="s")
    pooled = functools.partial(
        pl.kernel,
        out_type=jax.ShapeDtypeStruct((B, D), jnp.float32),
        mesh=mesh,
        compiler_params=pltpu.CompilerParams(use_tc_tiling_on_sc=False),
        scratch_types=[
            pltpu.VMEM((2 * B_PER_W, HALF_S), jnp.int32),
            pltpu.VMEM((S, D), jnp.float32),
            pltpu.VMEM((S, D), jnp.float32),
            pltpu.VMEM((B_PER_W, D), jnp.float32),
            pltpu.SemaphoreType.DMA,
            pltpu.SemaphoreType.DMA,
        ],
    )(_sc_body)(xT, emb_table)

    def _fc_body(p_ref, w_ref, b_ref, o_ref):
        o_ref[...] = (
            jnp.dot(p_ref[...], w_ref[...],
                    preferred_element_type=jnp.float32)
            + b_ref[...]
        )

    return pl.pallas_call(
        _fc_body,
        out_shape=jax.ShapeDtypeStruct((B, O), jnp.float32),
    )(pooled, w_scaled, b_row)


def kernel(x, emb_table, fc_w, fc_b):
    xT = x.astype(jnp.int32).T.reshape(2 * B, HALF_S)
    w_scaled = (fc_w.astype(jnp.float32) / S).T
    b_row = fc_b.astype(jnp.float32)[None, :]
    return _fast_text(xT, emb_table, w_scaled, b_row)
